# Initial kernel scaffold; baseline (speedup 1.0000x reference)
import jax
import jax.numpy as jnp
from jax.experimental import pallas as pl


def kernel(text, table, w_lin, b_lin):
    raise NotImplementedError("write your pallas kernel here")


# speedup vs baseline: 9.3002x; 9.3002x over previous
import dataclasses
import functools

import jax
import jax.numpy as jnp
from jax import lax
from jax.experimental import pallas as pl
from jax.experimental.pallas import tpu as pltpu
from jax.experimental.pallas import tpu_sc as plsc

V = 100000
D = 300
L = 50
B = 16384

NC = 2
NS = 16
NW = NC * NS
CB = B // NW
LANES = 16


def _tc_project_body(tb_ref, w_ref, o_ref):
    o_ref[0, 0, :] = jnp.sum(tb_ref[...] * w_ref[...], axis=1) * (1.0 / L)


def _tc_project(table, w_row):
    bv = 2000
    grid = V // bv
    out2d = pl.pallas_call(
        _tc_project_body,
        grid=(grid,),
        in_specs=[
            pl.BlockSpec((bv, D), lambda i: (i, 0)),
            pl.BlockSpec((1, D), lambda i: (0, 0)),
        ],
        out_specs=pl.BlockSpec((1, 1, bv), lambda i: (i, 0, 0)),
        out_shape=jax.ShapeDtypeStruct((grid, 1, bv), jnp.float32),
    )(table, w_row)
    return out2d.reshape(V)


_SC_MESH = plsc.VectorSubcoreMesh(core_axis_name="c", subcore_axis_name="s")

_SC_PARAMS = pltpu.CompilerParams()
if "needs_layout_passes" in pltpu.CompilerParams.__dataclass_fields__:
    _SC_PARAMS = dataclasses.replace(_SC_PARAMS, needs_layout_passes=False)


@functools.partial(
    pl.kernel,
    out_type=jax.ShapeDtypeStruct((B,), jnp.float32),
    mesh=_SC_MESH,
    scratch_types=[
        pltpu.VMEM((V,), jnp.float32),
        pltpu.VMEM((L, CB), jnp.int32),
        pltpu.VMEM((CB,), jnp.float32),
        pltpu.VMEM((LANES,), jnp.float32),
        pltpu.SemaphoreType.DMA,
        pltpu.SemaphoreType.DMA,
    ],
    compiler_params=_SC_PARAMS,
)
def _sc_gather_mean(t_hbm, text_hbm, bias_hbm, out_hbm,
                    t_v, idx_v, acc_v, b_v, sem_t, sem_i):
    wid = lax.axis_index("s") * NC + lax.axis_index("c")
    base = wid * CB
    cp_t = pltpu.async_copy(t_hbm, t_v, sem_t)
    cp_i = pltpu.async_copy(text_hbm.at[:, pl.ds(base, CB)], idx_v, sem_i)
    pltpu.sync_copy(bias_hbm, b_v)
    bias = b_v[...]
    cp_t.wait()
    cp_i.wait()

    @pl.loop(0, CB, step=LANES)
    def _(c):
        def body(l, acc):
            idx = idx_v[l, pl.ds(c, LANES)]
            return acc + plsc.load_gather(t_v, [idx])

        acc_v[pl.ds(c, LANES)] = lax.fori_loop(0, L, body, bias)

    pltpu.sync_copy(acc_v, out_hbm.at[pl.ds(base, CB)])


def kernel(text, table, w_lin, b_lin):
    text_i = text.astype(jnp.int32)
    w_row = w_lin.reshape(1, D).astype(jnp.float32)
    t = _tc_project(table, w_row)
    bias16 = jnp.broadcast_to(b_lin.astype(jnp.float32), (LANES,))
    out = _sc_gather_mean(t, text_i, bias16)
    return out.reshape(B, 1)
